# baseline (speedup 1.0000x reference)
import functools

import jax
import jax.numpy as jnp
from jax import lax
from jax.experimental import pallas as pl
from jax.experimental.pallas import tpu as pltpu
from jax.experimental.pallas import tpu_sc as plsc

_B = 10
_NPER = 1000
_K0 = 100
_NEG = 0.2
_EPS = 1e-15

_N = _B * _NPER
_E = 160000
_NW = 32
_REAL = _E // _NW
_EPW = 5120
_CH = 128
_NCH = _EPW // _CH
_NPC = 5000
_NPADC = 5120
_RPW = _NPADC // 16
_EPG = _E // _B


def _gat_sc_body(dp, xl_h, xr_h, src_h, dst_h, att_h, bias_h, out_h,
                 src_v, dst_v, dlb_a, dlb_b, l0, r0, l1, r1, exbuf, att_v,
                 bias_v, s16f, den_loc, dstage, outbuf,
                 g0l, g0r, g1l, g1r, ss0, ss1,
                 acc_s, den_s):
    c = lax.axis_index("c")
    s = lax.axis_index("s")
    wid = c * 16 + s
    base_nodes = c * _NPC
    iota = lax.iota(jnp.int32, 16)
    zero16 = jnp.zeros((16,), jnp.float32)
    nt = dp // 16

    pltpu.sync_copy(src_h.at[pl.ds(wid * _EPW, _EPW)], src_v)
    pltpu.sync_copy(dst_h.at[pl.ds(wid * _EPW, _EPW)], dst_v)
    pltpu.sync_copy(att_h, att_v)
    pltpu.sync_copy(bias_h, bias_v)

    def _z1(i, _):
        for t in range(nt):
            outbuf[i, pl.ds(t * 16, 16)] = zero16
        return 0
    lax.fori_loop(0, 16, _z1, 0)

    def _zc(b, _):
        pltpu.sync_copy(outbuf, acc_s.at[pl.ds(s * _RPW + b * 16, 16)])
        return 0
    lax.fori_loop(0, _RPW // 16, _zc, 0)

    def _zd(i, _):
        den_loc[pl.ds(i * 16, 16)] = zero16
        return 0
    lax.fori_loop(0, _NPADC // 16, _zd, 0)

    def _half(k, lbuf, rbuf, dlb):
        def _grp(g, _):
            p = k * _CH + g * 16

            def _edge(le, _):
                e = g * 16 + le
                acc = zero16
                for t in range(nt):
                    u = lbuf[e, pl.ds(t * 16, 16)] + rbuf[e, pl.ds(t * 16, 16)]
                    acc = acc + (att_v[pl.ds(t * 16, 16)]
                                 * jnp.maximum(u, _NEG * u))
                s16f[pl.ds(le * 16, 16)] = acc
                return 0
            lax.fori_loop(0, 16, _edge, 0)
            logit = zero16
            for d in range(16):
                logit = logit + plsc.load_gather(s16f, [iota * 16 + d])
            pos = p + iota
            ex = jnp.where(pos < _REAL, jnp.exp(logit), 0.0)
            exbuf[pl.ds(g * 16, 16)] = ex
            dl = dst_v[pl.ds(p, 16)] - base_nodes
            dlb[pl.ds(g * 16, 16)] = dl
            plsc.addupdate_scatter(den_loc, [dl], ex)

            def _sc(le, _):
                e = g * 16 + le
                scv = plsc.load_gather(exbuf,
                                       [jnp.zeros((16,), jnp.int32) + e])
                for t in range(nt):
                    blk = lbuf[e, pl.ds(t * 16, 16)]
                    lbuf[e, pl.ds(t * 16, 16)] = blk * scv
                return 0
            return lax.fori_loop(0, 16, _sc, 0)
        lax.fori_loop(0, _CH // 16, _grp, 0)

    def _round(j, _):
        a = 2 * j
        b = 2 * j + 1
        da_l = pltpu.async_copy(xl_h.at[src_v.at[pl.ds(a * _CH, _CH)]],
                                l0, g0l)
        da_r = pltpu.async_copy(xr_h.at[dst_v.at[pl.ds(a * _CH, _CH)]],
                                r0, g0r)
        db_l = pltpu.async_copy(xl_h.at[src_v.at[pl.ds(b * _CH, _CH)]],
                                l1, g1l)
        db_r = pltpu.async_copy(xr_h.at[dst_v.at[pl.ds(b * _CH, _CH)]],
                                r1, g1r)
        da_l.wait()
        da_r.wait()
        _half(a, l0, r0, dlb_a)
        dsa = pltpu.async_copy(l0, acc_s.at[dlb_a], ss0, add=True)
        db_l.wait()
        db_r.wait()
        _half(b, l1, r1, dlb_b)
        dsb = pltpu.async_copy(l1, acc_s.at[dlb_b], ss1, add=True)
        dsa.wait()
        dsb.wait()
        return 0
    lax.fori_loop(0, _NCH // 2, _round, 0)

    pltpu.sync_copy(den_loc, den_s.at[pl.ds(s * _NPADC, _NPADC)])
    plsc.subcore_barrier()

    def _zd2(i, _):
        den_loc[pl.ds(i * 16, 16)] = zero16
        return 0
    lax.fori_loop(0, _RPW // 16, _zd2, 0)

    def _slot(w, _):
        pltpu.sync_copy(den_s.at[pl.ds(w * _NPADC + s * _RPW, _RPW)], dstage)

        def _add(i, _):
            den_loc[pl.ds(i * 16, 16)] = (den_loc[pl.ds(i * 16, 16)]
                                          + dstage[pl.ds(i * 16, 16)])
            return 0
        return lax.fori_loop(0, _RPW // 16, _add, 0)
    lax.fori_loop(0, 16, _slot, 0)

    def _blk(b, _):
        nb = s * _RPW + b * 16
        pltpu.sync_copy(acc_s.at[pl.ds(nb, 16)], outbuf)

        def _node(n, _):
            lnv = jnp.zeros((16,), jnp.int32) + (b * 16 + n)
            dv = plsc.load_gather(den_loc, [lnv])
            scv = 1.0 / jnp.maximum(dv, 1e-16)
            for t in range(nt):
                outbuf[n, pl.ds(t * 16, 16)] = (
                    outbuf[n, pl.ds(t * 16, 16)] * scv
                    + bias_v[pl.ds(t * 16, 16)])
            return 0
        lax.fori_loop(0, 16, _node, 0)
        pltpu.sync_copy(outbuf, out_h.at[c, pl.ds(nb, 16)])
        return 0
    lax.fori_loop(0, _RPW // 16, _blk, 0)


@functools.cache
def _make_gat_sc(dp):
    mesh = plsc.VectorSubcoreMesh(core_axis_name="c", subcore_axis_name="s",
                                  num_cores=2, num_subcores=16)
    return functools.partial(
        pl.kernel,
        out_type=jax.ShapeDtypeStruct((2, _NPADC, dp), jnp.float32),
        mesh=mesh,
        compiler_params=pltpu.CompilerParams(needs_layout_passes=False),
        scratch_types=[
            pltpu.VMEM((_EPW,), jnp.int32),
            pltpu.VMEM((_EPW,), jnp.int32),
            pltpu.VMEM((_CH,), jnp.int32),
            pltpu.VMEM((_CH,), jnp.int32),
            pltpu.VMEM((_CH, dp), jnp.float32),
            pltpu.VMEM((_CH, dp), jnp.float32),
            pltpu.VMEM((_CH, dp), jnp.float32),
            pltpu.VMEM((_CH, dp), jnp.float32),
            pltpu.VMEM((_CH,), jnp.float32),
            pltpu.VMEM((dp,), jnp.float32),
            pltpu.VMEM((dp,), jnp.float32),
            pltpu.VMEM((256,), jnp.float32),
            pltpu.VMEM((_NPADC,), jnp.float32),
            pltpu.VMEM((_RPW,), jnp.float32),
            pltpu.VMEM((16, dp), jnp.float32),
            pltpu.SemaphoreType.DMA,
            pltpu.SemaphoreType.DMA,
            pltpu.SemaphoreType.DMA,
            pltpu.SemaphoreType.DMA,
            pltpu.SemaphoreType.DMA,
            pltpu.SemaphoreType.DMA,
            pltpu.VMEM_SHARED((_NPADC, dp), jnp.float32),
            pltpu.VMEM_SHARED((16 * _NPADC,), jnp.float32),
        ],
    )(functools.partial(_gat_sc_body, dp))


def _proj_body(x_ref, w_ref, b_ref, o_ref):
    o_ref[...] = jnp.dot(x_ref[...], w_ref[...],
                         preferred_element_type=jnp.float32) + b_ref[...]


def _proj(x, w, b):
    n, din = x.shape
    dout = w.shape[1]
    blk = 1000
    return pl.pallas_call(
        _proj_body,
        grid=(n // blk,),
        in_specs=[pl.BlockSpec((blk, din), lambda i: (i, 0)),
                  pl.BlockSpec((din, dout), lambda i: (0, 0)),
                  pl.BlockSpec((1, dout), lambda i: (0, 0))],
        out_specs=pl.BlockSpec((blk, dout), lambda i: (i, 0)),
        out_shape=jax.ShapeDtypeStruct((n, dout), jnp.float32),
    )(x, w, b.reshape(1, -1))


def _pad_cols(a, dp):
    if a.shape[-1] == dp:
        return a
    return jnp.pad(a, [(0, 0)] * (a.ndim - 1) + [(0, dp - a.shape[-1])])


def _gat_sparse_sc(x, srcp, dstp, p, dout):
    dp = 128
    w2 = jnp.concatenate([_pad_cols(p["Wl"], dp), _pad_cols(p["Wr"], dp)],
                         axis=1)
    b2 = jnp.concatenate([_pad_cols(p["bl"], dp), _pad_cols(p["br"], dp)])
    xlr = _proj(x, w2, b2)
    xl, xr = xlr[:, :dp], xlr[:, dp:]
    att = _pad_cols(p["att"], dp)
    bias = _pad_cols(p["bias"], dp)
    out2 = _make_gat_sc(dp)(xl, xr, srcp, dstp, att, bias)
    out = jnp.concatenate([out2[0, :_NPC], out2[1, :_NPC]], axis=0)
    return out[:, :dout]


def _gat_dense(z, p):
    xl = z @ p["Wl"] + p["bl"]
    xr = z @ p["Wr"] + p["br"]
    e = xl[:, None, :, :] + xr[:, :, None, :]
    e = jnp.where(e >= 0, e, _NEG * e)
    logits = jnp.einsum("bijd,d->bij", e, p["att"])
    alpha = jax.nn.softmax(logits, axis=-1)
    return jnp.einsum("bij,bjd->bid", alpha, xl) + p["bias"]


def _pool_body(s_ref, z_ref, adj_ref, x1_ref, sc_ref):
    s = s_ref[...]
    z = z_ref[...]
    adj = adj_ref[0]
    sm = jnp.exp(s - jnp.max(s, axis=-1, keepdims=True))
    smx = jnp.sum(sm, axis=-1, keepdims=True)
    sft = sm / smx
    x1 = jnp.dot(sft.T, z, preferred_element_type=jnp.float32)
    x1_ref[0] = x1
    a_s = jnp.dot(adj, sft, preferred_element_type=jnp.float32)
    tr = jnp.sum(a_s * sft)
    gram = jnp.dot(sft.T, sft, preferred_element_type=jnp.float32)
    gsq = jnp.sum(gram * gram)
    ssq = jnp.sum(adj * adj)
    ent = jnp.sum(-sft * jnp.log(sft + _EPS))
    col = lax.broadcasted_iota(jnp.int32, (1, 128), 1)
    row0 = (jnp.where(col == 0, ssq, 0.0) + jnp.where(col == 1, tr, 0.0)
            + jnp.where(col == 2, gsq, 0.0) + jnp.where(col == 3, ent, 0.0))
    sc_ref[0, 0:1, :] = row0
    sc_ref[0, 1:2, :] = jnp.sum(x1, axis=0, keepdims=True) * (1.0 / _K0)


def _pool_losses(s, z, adj):
    x1, scal = pl.pallas_call(
        _pool_body,
        grid=(_B,),
        in_specs=[
            pl.BlockSpec((_NPER, _K0), lambda i: (i, 0)),
            pl.BlockSpec((_NPER, 128), lambda i: (i, 0)),
            pl.BlockSpec((1, _NPER, _NPER), lambda i: (i, 0, 0)),
        ],
        out_specs=[
            pl.BlockSpec((1, _K0, 128), lambda i: (i, 0, 0)),
            pl.BlockSpec((1, 2, 128), lambda i: (i, 0, 0)),
        ],
        out_shape=[
            jax.ShapeDtypeStruct((_B, _K0, 128), jnp.float32),
            jax.ShapeDtypeStruct((_B, 2, 128), jnp.float32),
        ],
    )(s.reshape(_B * _NPER, _K0), z.reshape(_B * _NPER, 128), adj)
    return x1, scal


def _dense_gat_body(p0, p1, x1_ref, w_ref, b_ref, a_ref, x2_ref):
    z = x1_ref[0]
    for i, (po, pa) in enumerate((p0, p1)):
        wl = w_ref[i * 2]
        wr = w_ref[i * 2 + 1]
        xl = jnp.dot(z, wl, preferred_element_type=jnp.float32) + b_ref[
            i * 3, :][None, :]
        xr = jnp.dot(z, wr, preferred_element_type=jnp.float32) + b_ref[
            i * 3 + 1, :][None, :]
        e = xl[None, :, :] + xr[:, None, :]
        e = jnp.maximum(e, _NEG * e)
        logits = jnp.dot(e.reshape(_K0 * _K0, 128), a_ref[i, :],
                         preferred_element_type=jnp.float32)
        logits = logits.reshape(_K0, _K0)
        ex = jnp.exp(logits - jnp.max(logits, axis=-1, keepdims=True))
        alpha = ex / jnp.sum(ex, axis=-1, keepdims=True)
        z = jnp.dot(alpha, xl, preferred_element_type=jnp.float32) + b_ref[
            i * 3 + 2, :][None, :]
    x2_ref[0] = jnp.sum(z, axis=0, keepdims=True)


def _dense_gats(x1, ps):
    w = jnp.stack([ps[0]["Wl"], ps[0]["Wr"], ps[1]["Wl"], ps[1]["Wr"]])
    b = jnp.stack([ps[0]["bl"], ps[0]["br"], ps[0]["bias"],
                   ps[1]["bl"], ps[1]["br"], ps[1]["bias"]])
    a = jnp.stack([ps[0]["att"], ps[1]["att"]])
    return pl.pallas_call(
        functools.partial(_dense_gat_body, (0, 0), (1, 1)),
        grid=(_B,),
        in_specs=[
            pl.BlockSpec((1, _K0, 128), lambda i: (i, 0, 0)),
            pl.BlockSpec((4, 128, 128), lambda i: (0, 0, 0)),
            pl.BlockSpec((6, 128), lambda i: (0, 0)),
            pl.BlockSpec((2, 128), lambda i: (0, 0)),
        ],
        out_specs=pl.BlockSpec((1, 1, 128), lambda i: (i, 0, 0)),
        out_shape=jax.ShapeDtypeStruct((_B, 1, 128), jnp.float32),
    )(x1, w, b, a)


def _heads_body(x_ref, xm_ref, w_ref, wm_ref, o_ref):
    o_ref[...] = (jnp.dot(x_ref[...], w_ref[...],
                          preferred_element_type=jnp.float32)
                  + jnp.dot(xm_ref[...], wm_ref[...],
                            preferred_element_type=jnp.float32))


def _final_heads(x2, xm, params):
    h0, h1 = params["heads"][0], params["heads"][1]
    xp = jnp.zeros((16, 128), jnp.float32).at[:_B, :].set(x2)
    xmp = jnp.zeros((16, 128), jnp.float32).at[:_B, :].set(xm)
    wp = jnp.zeros((128, 128), jnp.float32)
    wp = wp.at[:, 0:2].set(params["lin"]["W"]).at[:, 2:4].set(h1["W"])
    wmp = jnp.zeros((128, 128), jnp.float32).at[:, 4:6].set(h0["W"])
    o = pl.pallas_call(
        _heads_body,
        out_shape=jax.ShapeDtypeStruct((16, 128), jnp.float32),
    )(xp, xmp, wp, wmp)
    out = o[:_B, 0:2] + params["lin"]["b"]
    c1 = o[:_B, 2:4] + h1["b"]
    c0 = o[:_B, 4:6] + h0["b"]
    return out, c0, c1


def kernel(x, edge_index, batch, params):
    del batch
    src, dst = edge_index[0], edge_index[1]

    fill = jnp.repeat(jnp.array([0, _NPC], jnp.int32), 16)[:, None]
    col_ok = jnp.arange(_EPW, dtype=jnp.int32)[None, :] < _REAL
    srcp = jnp.where(col_ok, jnp.pad(src.reshape(_NW, _REAL),
                                     ((0, 0), (0, _EPW - _REAL))),
                     fill).reshape(-1)
    dstp = jnp.where(col_ok, jnp.pad(dst.reshape(_NW, _REAL),
                                     ((0, 0), (0, _EPW - _REAL))),
                     fill).reshape(-1)

    s = x
    for p, dout in zip(params["pool"][0], (64, 100)):
        s = _gat_sparse_sc(s, srcp, dstp, p, dout)
    z = x
    for p, dout in zip(params["embed"][0], (128, 128)):
        z = _gat_sparse_sc(z, srcp, dstp, p, dout)

    b_e = src // _NPER
    adj = jnp.zeros((_B, _NPER, _NPER), jnp.float32).at[
        b_e, src - b_e * _NPER, dst - b_e * _NPER].add(1.0)

    x1, scal = _pool_losses(s, z, adj)
    ssq_adj = jnp.sum(scal[:, 0, 0])
    tr = jnp.sum(scal[:, 0, 1])
    gsq = jnp.sum(scal[:, 0, 2])
    ent = jnp.sum(scal[:, 0, 3])
    ll0 = jnp.sqrt(ssq_adj - 2.0 * tr + gsq) / (_B * _NPER * _NPER)
    el0 = ent / (_B * _NPER)
    xm = scal[:, 1, :]

    x2 = _dense_gats(x1, params["embed"][1])[:, 0, :]

    ll1 = jnp.float32(0.0)
    el1 = jnp.float32(0.0)

    out, c0, c1 = _final_heads(x2, xm, params)
    return (out, ll0, ll1, el0, el1, c0, c1)


# speedup vs baseline: 1.0446x; 1.0446x over previous
import functools

import jax
import jax.numpy as jnp
from jax import lax
from jax.experimental import pallas as pl
from jax.experimental.pallas import tpu as pltpu
from jax.experimental.pallas import tpu_sc as plsc

_B = 10
_NPER = 1000
_K0 = 100
_NEG = 0.2
_EPS = 1e-15

_N = _B * _NPER
_E = 160000
_NW = 32
_REAL = _E // _NW
_EPW = 5120
_CH = 128
_NCH = _EPW // _CH
_NPC = 5000
_NPADC = 5120
_RPW = _NPADC // 16
_EPG = _E // _B


def _gat_sc_body(dp, xl_h, xr_h, src_h, dst_h, att_h, bias_h, out_h,
                 src_v, dst_v, dlb_a, dlb_b, l0, r0, l1, r1, exbuf, att_v,
                 bias_v, s16f, den_loc, dstage, outbuf,
                 g0l, g0r, g1l, g1r, ss0, ss1,
                 acc_s, den_s):
    c = lax.axis_index("c")
    s = lax.axis_index("s")
    wid = c * 16 + s
    base_nodes = c * _NPC
    iota = lax.iota(jnp.int32, 16)
    zero16 = jnp.zeros((16,), jnp.float32)
    nt = dp // 16

    pltpu.sync_copy(src_h.at[pl.ds(wid * _EPW, _EPW)], src_v)
    pltpu.sync_copy(dst_h.at[pl.ds(wid * _EPW, _EPW)], dst_v)
    pltpu.sync_copy(att_h, att_v)
    pltpu.sync_copy(bias_h, bias_v)

    def _z1(i, _):
        for t in range(nt):
            outbuf[i, pl.ds(t * 16, 16)] = zero16
        return 0
    lax.fori_loop(0, 16, _z1, 0)

    def _zc(b, _):
        pltpu.sync_copy(outbuf, acc_s.at[pl.ds(s * _RPW + b * 16, 16)])
        return 0
    lax.fori_loop(0, _RPW // 16, _zc, 0)

    def _zd(i, _):
        den_loc[pl.ds(i * 16, 16)] = zero16
        return 0
    lax.fori_loop(0, _NPADC // 16, _zd, 0)

    def _half(k, lbuf, rbuf, dlb):
        def _grp(g, _):
            p = k * _CH + g * 16

            def _edge(le, _):
                e = g * 16 + le
                acc = zero16
                for t in range(nt):
                    u = lbuf[e, pl.ds(t * 16, 16)] + rbuf[e, pl.ds(t * 16, 16)]
                    acc = acc + (att_v[pl.ds(t * 16, 16)]
                                 * jnp.maximum(u, _NEG * u))
                s16f[pl.ds(le * 16, 16)] = acc
                return 0
            lax.fori_loop(0, 16, _edge, 0)
            logit = zero16
            for d in range(16):
                logit = logit + plsc.load_gather(s16f, [iota * 16 + d])
            pos = p + iota
            ex = jnp.where(pos < _REAL, jnp.exp(logit), 0.0)
            exbuf[pl.ds(g * 16, 16)] = ex
            dl = dst_v[pl.ds(p, 16)] - base_nodes
            dlb[pl.ds(g * 16, 16)] = dl
            plsc.addupdate_scatter(den_loc, [dl], ex)

            def _sc(le, _):
                e = g * 16 + le
                scv = plsc.load_gather(exbuf,
                                       [jnp.zeros((16,), jnp.int32) + e])
                for t in range(nt):
                    blk = lbuf[e, pl.ds(t * 16, 16)]
                    lbuf[e, pl.ds(t * 16, 16)] = blk * scv
                return 0
            return lax.fori_loop(0, 16, _sc, 0)
        lax.fori_loop(0, _CH // 16, _grp, 0)

    def _round(j, _):
        a = 2 * j
        b = 2 * j + 1
        da_l = pltpu.async_copy(xl_h.at[src_v.at[pl.ds(a * _CH, _CH)]],
                                l0, g0l)
        da_r = pltpu.async_copy(xr_h.at[dst_v.at[pl.ds(a * _CH, _CH)]],
                                r0, g0r)
        db_l = pltpu.async_copy(xl_h.at[src_v.at[pl.ds(b * _CH, _CH)]],
                                l1, g1l)
        db_r = pltpu.async_copy(xr_h.at[dst_v.at[pl.ds(b * _CH, _CH)]],
                                r1, g1r)
        da_l.wait()
        da_r.wait()
        _half(a, l0, r0, dlb_a)
        dsa = pltpu.async_copy(l0, acc_s.at[dlb_a], ss0, add=True)
        db_l.wait()
        db_r.wait()
        _half(b, l1, r1, dlb_b)
        dsb = pltpu.async_copy(l1, acc_s.at[dlb_b], ss1, add=True)
        dsa.wait()
        dsb.wait()
        return 0
    lax.fori_loop(0, _NCH // 2, _round, 0)

    pltpu.sync_copy(den_loc, den_s.at[pl.ds(s * _NPADC, _NPADC)])
    plsc.subcore_barrier()

    def _zd2(i, _):
        den_loc[pl.ds(i * 16, 16)] = zero16
        return 0
    lax.fori_loop(0, _RPW // 16, _zd2, 0)

    def _slot(w, _):
        pltpu.sync_copy(den_s.at[pl.ds(w * _NPADC + s * _RPW, _RPW)], dstage)

        def _add(i, _):
            den_loc[pl.ds(i * 16, 16)] = (den_loc[pl.ds(i * 16, 16)]
                                          + dstage[pl.ds(i * 16, 16)])
            return 0
        return lax.fori_loop(0, _RPW // 16, _add, 0)
    lax.fori_loop(0, 16, _slot, 0)

    def _blk(b, _):
        nb = s * _RPW + b * 16
        pltpu.sync_copy(acc_s.at[pl.ds(nb, 16)], outbuf)

        def _node(n, _):
            lnv = jnp.zeros((16,), jnp.int32) + (b * 16 + n)
            dv = plsc.load_gather(den_loc, [lnv])
            scv = 1.0 / jnp.maximum(dv, 1e-16)
            for t in range(nt):
                outbuf[n, pl.ds(t * 16, 16)] = (
                    outbuf[n, pl.ds(t * 16, 16)] * scv
                    + bias_v[pl.ds(t * 16, 16)])
            return 0
        lax.fori_loop(0, 16, _node, 0)
        pltpu.sync_copy(outbuf, out_h.at[c, pl.ds(nb, 16)])
        return 0
    lax.fori_loop(0, _RPW // 16, _blk, 0)


@functools.cache
def _make_gat_sc(dp):
    mesh = plsc.VectorSubcoreMesh(core_axis_name="c", subcore_axis_name="s",
                                  num_cores=2, num_subcores=16)
    return functools.partial(
        pl.kernel,
        out_type=jax.ShapeDtypeStruct((2, _NPADC, dp), jnp.float32),
        mesh=mesh,
        compiler_params=pltpu.CompilerParams(needs_layout_passes=False),
        scratch_types=[
            pltpu.VMEM((_EPW,), jnp.int32),
            pltpu.VMEM((_EPW,), jnp.int32),
            pltpu.VMEM((_CH,), jnp.int32),
            pltpu.VMEM((_CH,), jnp.int32),
            pltpu.VMEM((_CH, dp), jnp.float32),
            pltpu.VMEM((_CH, dp), jnp.float32),
            pltpu.VMEM((_CH, dp), jnp.float32),
            pltpu.VMEM((_CH, dp), jnp.float32),
            pltpu.VMEM((_CH,), jnp.float32),
            pltpu.VMEM((dp,), jnp.float32),
            pltpu.VMEM((dp,), jnp.float32),
            pltpu.VMEM((256,), jnp.float32),
            pltpu.VMEM((_NPADC,), jnp.float32),
            pltpu.VMEM((_RPW,), jnp.float32),
            pltpu.VMEM((16, dp), jnp.float32),
            pltpu.SemaphoreType.DMA,
            pltpu.SemaphoreType.DMA,
            pltpu.SemaphoreType.DMA,
            pltpu.SemaphoreType.DMA,
            pltpu.SemaphoreType.DMA,
            pltpu.SemaphoreType.DMA,
            pltpu.VMEM_SHARED((_NPADC, dp), jnp.float32),
            pltpu.VMEM_SHARED((16 * _NPADC,), jnp.float32),
        ],
    )(functools.partial(_gat_sc_body, dp))


def _proj_body(x_ref, w_ref, b_ref, o_ref):
    o_ref[...] = jnp.dot(x_ref[...], w_ref[...],
                         preferred_element_type=jnp.float32) + b_ref[...]


def _proj(x, w, b):
    n, din = x.shape
    dout = w.shape[1]
    blk = 1000
    return pl.pallas_call(
        _proj_body,
        grid=(n // blk,),
        in_specs=[pl.BlockSpec((blk, din), lambda i: (i, 0)),
                  pl.BlockSpec((din, dout), lambda i: (0, 0)),
                  pl.BlockSpec((1, dout), lambda i: (0, 0))],
        out_specs=pl.BlockSpec((blk, dout), lambda i: (i, 0)),
        out_shape=jax.ShapeDtypeStruct((n, dout), jnp.float32),
    )(x, w, b.reshape(1, -1))


def _pad_cols(a, dp):
    if a.shape[-1] == dp:
        return a
    return jnp.pad(a, [(0, 0)] * (a.ndim - 1) + [(0, dp - a.shape[-1])])


def _gat_sparse_sc(x, srcp, dstp, p, dout):
    dp = 128
    w2 = jnp.concatenate([_pad_cols(p["Wl"], dp), _pad_cols(p["Wr"], dp)],
                         axis=1)
    b2 = jnp.concatenate([_pad_cols(p["bl"], dp), _pad_cols(p["br"], dp)])
    xlr = _proj(x, w2, b2)
    xl, xr = xlr[:, :dp], xlr[:, dp:]
    att = _pad_cols(p["att"], dp)
    bias = _pad_cols(p["bias"], dp)
    out2 = _make_gat_sc(dp)(xl, xr, srcp, dstp, att, bias)
    out = jnp.concatenate([out2[0, :_NPC], out2[1, :_NPC]], axis=0)
    return out[:, :dout]


def _gat_dense(z, p):
    xl = z @ p["Wl"] + p["bl"]
    xr = z @ p["Wr"] + p["br"]
    e = xl[:, None, :, :] + xr[:, :, None, :]
    e = jnp.where(e >= 0, e, _NEG * e)
    logits = jnp.einsum("bijd,d->bij", e, p["att"])
    alpha = jax.nn.softmax(logits, axis=-1)
    return jnp.einsum("bij,bjd->bid", alpha, xl) + p["bias"]


def _pool_body(s_ref, z_ref, adj_ref, x1_ref, sc_ref):
    s = s_ref[...]
    z = z_ref[...]
    adj = adj_ref[0]
    sm = jnp.exp(s - jnp.max(s, axis=-1, keepdims=True))
    smx = jnp.sum(sm, axis=-1, keepdims=True)
    sft = sm / smx
    x1 = jnp.dot(sft.T, z, preferred_element_type=jnp.float32)
    x1_ref[0] = x1
    a_s = jnp.dot(adj, sft, preferred_element_type=jnp.float32)
    tr = jnp.sum(a_s * sft)
    gram = jnp.dot(sft.T, sft, preferred_element_type=jnp.float32)
    gsq = jnp.sum(gram * gram)
    ssq = jnp.sum(adj * adj)
    ent = jnp.sum(-sft * jnp.log(sft + _EPS))
    col = lax.broadcasted_iota(jnp.int32, (1, 128), 1)
    row0 = (jnp.where(col == 0, ssq, 0.0) + jnp.where(col == 1, tr, 0.0)
            + jnp.where(col == 2, gsq, 0.0) + jnp.where(col == 3, ent, 0.0))
    sc_ref[0, 0:1, :] = row0
    sc_ref[0, 1:2, :] = jnp.sum(x1, axis=0, keepdims=True) * (1.0 / _K0)


def _pool_losses(s, z, adj):
    x1, scal = pl.pallas_call(
        _pool_body,
        grid=(_B,),
        in_specs=[
            pl.BlockSpec((_NPER, _K0), lambda i: (i, 0)),
            pl.BlockSpec((_NPER, 128), lambda i: (i, 0)),
            pl.BlockSpec((1, _NPER, _NPER), lambda i: (i, 0, 0)),
        ],
        out_specs=[
            pl.BlockSpec((1, _K0, 128), lambda i: (i, 0, 0)),
            pl.BlockSpec((1, 2, 128), lambda i: (i, 0, 0)),
        ],
        out_shape=[
            jax.ShapeDtypeStruct((_B, _K0, 128), jnp.float32),
            jax.ShapeDtypeStruct((_B, 2, 128), jnp.float32),
        ],
    )(s.reshape(_B * _NPER, _K0), z.reshape(_B * _NPER, 128), adj)
    return x1, scal


def _dense_gat_body(p0, p1, x1_ref, w_ref, b_ref, a_ref, x2_ref):
    z = x1_ref[0]
    for i, (po, pa) in enumerate((p0, p1)):
        wl = w_ref[i * 2]
        wr = w_ref[i * 2 + 1]
        xl = jnp.dot(z, wl, preferred_element_type=jnp.float32,
                     precision=lax.Precision.HIGHEST) + b_ref[
            i * 3, :][None, :]
        xr = jnp.dot(z, wr, preferred_element_type=jnp.float32,
                     precision=lax.Precision.HIGHEST) + b_ref[
            i * 3 + 1, :][None, :]
        e = xl[None, :, :] + xr[:, None, :]
        e = jnp.maximum(e, _NEG * e)
        logits = jnp.sum(e * a_ref[i, :][None, None, :], axis=-1)
        ex = jnp.exp(logits - jnp.max(logits, axis=-1, keepdims=True))
        alpha = ex / jnp.sum(ex, axis=-1, keepdims=True)
        z = jnp.dot(alpha, xl, preferred_element_type=jnp.float32,
                    precision=lax.Precision.HIGHEST) + b_ref[
            i * 3 + 2, :][None, :]
    x2_ref[0] = jnp.sum(z, axis=0, keepdims=True)


def _dense_gats(x1, ps):
    w = jnp.stack([ps[0]["Wl"], ps[0]["Wr"], ps[1]["Wl"], ps[1]["Wr"]])
    b = jnp.stack([ps[0]["bl"], ps[0]["br"], ps[0]["bias"],
                   ps[1]["bl"], ps[1]["br"], ps[1]["bias"]])
    a = jnp.stack([ps[0]["att"], ps[1]["att"]])
    return pl.pallas_call(
        functools.partial(_dense_gat_body, (0, 0), (1, 1)),
        grid=(_B,),
        in_specs=[
            pl.BlockSpec((1, _K0, 128), lambda i: (i, 0, 0)),
            pl.BlockSpec((4, 128, 128), lambda i: (0, 0, 0)),
            pl.BlockSpec((6, 128), lambda i: (0, 0)),
            pl.BlockSpec((2, 128), lambda i: (0, 0)),
        ],
        out_specs=pl.BlockSpec((1, 1, 128), lambda i: (i, 0, 0)),
        out_shape=jax.ShapeDtypeStruct((_B, 1, 128), jnp.float32),
    )(x1, w, b, a)


def _heads_body(x_ref, xm_ref, w_ref, wm_ref, o_ref):
    o_ref[...] = (jnp.dot(x_ref[...], w_ref[...],
                          preferred_element_type=jnp.float32)
                  + jnp.dot(xm_ref[...], wm_ref[...],
                            preferred_element_type=jnp.float32))


def _final_heads(x2, xm, params):
    h0, h1 = params["heads"][0], params["heads"][1]
    xp = jnp.zeros((16, 128), jnp.float32).at[:_B, :].set(x2)
    xmp = jnp.zeros((16, 128), jnp.float32).at[:_B, :].set(xm)
    wp = jnp.zeros((128, 128), jnp.float32)
    wp = wp.at[:, 0:2].set(params["lin"]["W"]).at[:, 2:4].set(h1["W"])
    wmp = jnp.zeros((128, 128), jnp.float32).at[:, 4:6].set(h0["W"])
    o = pl.pallas_call(
        _heads_body,
        out_shape=jax.ShapeDtypeStruct((16, 128), jnp.float32),
    )(xp, xmp, wp, wmp)
    out = o[:_B, 0:2] + params["lin"]["b"]
    c1 = o[:_B, 2:4] + h1["b"]
    c0 = o[:_B, 4:6] + h0["b"]
    return out, c0, c1


def kernel(x, edge_index, batch, params):
    del batch
    src, dst = edge_index[0], edge_index[1]

    fill = jnp.repeat(jnp.array([0, _NPC], jnp.int32), 16)[:, None]
    col_ok = jnp.arange(_EPW, dtype=jnp.int32)[None, :] < _REAL
    srcp = jnp.where(col_ok, jnp.pad(src.reshape(_NW, _REAL),
                                     ((0, 0), (0, _EPW - _REAL))),
                     fill).reshape(-1)
    dstp = jnp.where(col_ok, jnp.pad(dst.reshape(_NW, _REAL),
                                     ((0, 0), (0, _EPW - _REAL))),
                     fill).reshape(-1)

    s = x
    for p, dout in zip(params["pool"][0], (64, 100)):
        s = _gat_sparse_sc(s, srcp, dstp, p, dout)
    z = x
    for p, dout in zip(params["embed"][0], (128, 128)):
        z = _gat_sparse_sc(z, srcp, dstp, p, dout)

    b_e = src // _NPER
    adj = jnp.zeros((_B, _NPER, _NPER), jnp.float32).at[
        b_e, src - b_e * _NPER, dst - b_e * _NPER].add(1.0)

    x1, scal = _pool_losses(s, z, adj)
    ssq_adj = jnp.sum(scal[:, 0, 0])
    tr = jnp.sum(scal[:, 0, 1])
    gsq = jnp.sum(scal[:, 0, 2])
    ent = jnp.sum(scal[:, 0, 3])
    ll0 = jnp.sqrt(ssq_adj - 2.0 * tr + gsq) / (_B * _NPER * _NPER)
    el0 = ent / (_B * _NPER)
    xm = scal[:, 1, :]

    x2 = _dense_gats(x1, params["embed"][1])[:, 0, :]

    ll1 = jnp.float32(0.0)
    el1 = jnp.float32(0.0)

    out, c0, c1 = _final_heads(x2, xm, params)
    return (out, ll0, ll1, el0, el1, c0, c1)
